# baseline (speedup 1.0000x reference)
import functools

import jax
import jax.numpy as jnp
from jax import lax
from jax.experimental import pallas as pl
from jax.experimental.pallas import tpu as pltpu
from jax.experimental.pallas import tpu_sc as plsc

N = 10000
E = 320000
D_IN = 128
D_HID = 16
D_OUT = 40
D_OUTP = 48

NC = 2
NS = 16
NW = NC * NS
EPW = E // NW
K = 100
NCH = EPW // K
NR = N // 16
RPT = NR // NS
RB = 16 * RPT
TAIL0 = NS * RB
TAILN = N - TAIL0

_MESH = plsc.VectorSubcoreMesh(core_axis_name="c", subcore_axis_name="s")
_SC_PARAMS = pltpu.CompilerParams(use_tc_tiling_on_sc=False,
                                  needs_layout_passes=False)


def _tile_rows_copy(src, dst, s):
    pltpu.sync_copy(src.at[pl.ds(s * RB, RB)], dst.at[pl.ds(s * RB, RB)])

    @pl.when(s == 0)
    def _():
        pltpu.sync_copy(src.at[pl.ds(TAIL0, TAILN)], dst.at[pl.ds(TAIL0, TAILN)])


def _rsqrt16(x):
    i = plsc.bitcast(x, jnp.int32)
    y = plsc.bitcast(jnp.int32(0x5F3759DF) - lax.shift_right_logical(i, 1),
                     jnp.float32)
    for _ in range(3):
        y = y * (1.5 - 0.5 * x * y * y)
    return y


def _edge_pipeline(val_ref, acc, src_v, dst_v, bufs, gsems, ssems):
    for u in range(4):
        pltpu.async_copy(val_ref.at[src_v.at[u]], bufs[u], gsems[u])

    @pl.loop(0, NCH, step=4)
    def _(j):
        for u in range(4):
            pltpu.make_async_copy(val_ref.at[src_v.at[j + u]], bufs[u],
                                  gsems[u]).wait()
            pltpu.async_copy(bufs[u], acc.at[dst_v.at[j + u]], ssems[u],
                             add=True)
        for u in range(4):
            @pl.when(j + 4 + u < NCH)
            def _(u=u):
                pltpu.make_async_copy(bufs[u], acc.at[dst_v.at[j + u]],
                                      ssems[u]).wait()
                pltpu.async_copy(val_ref.at[src_v.at[j + 4 + u]], bufs[u],
                                 gsems[u])

    for u in range(4):
        pltpu.make_async_copy(bufs[u], acc.at[dst_v.at[0]], ssems[u]).wait()


@functools.partial(
    pl.kernel,
    out_type=(
        jax.ShapeDtypeStruct((NC, N, D_HID), jnp.float32),
        jax.ShapeDtypeStruct((NC, N, D_HID), jnp.float32),
        jax.ShapeDtypeStruct((N,), jnp.float32),
    ),
    mesh=_MESH,
    scratch_types=[
        pltpu.VMEM((NCH, K), jnp.int32),
        pltpu.VMEM((NCH, K), jnp.int32),
        pltpu.VMEM((2, EPW), jnp.int32),
        pltpu.VMEM((NR, 16), jnp.float32),
        pltpu.VMEM((5, 125), jnp.int32),
        pltpu.VMEM((RPT + 1, 16), jnp.float32),
        pltpu.VMEM((RB + TAILN,), jnp.float32),
        pltpu.VMEM((RB, D_HID), jnp.float32),
        pltpu.VMEM((TAILN, D_HID), jnp.float32),
        [pltpu.VMEM((K, D_HID), jnp.float32)] * 4,
        pltpu.VMEM_SHARED((NR, 16), jnp.float32),
        pltpu.VMEM_SHARED((N, D_HID), jnp.float32),
        [pltpu.SemaphoreType.DMA] * 4,
        [pltpu.SemaphoreType.DMA] * 4,
    ],
    compiler_params=_SC_PARAMS,
)
def _sc_k1(xw_hbm, src_hbm, dst_hbm, dstf_hbm, iota_hbm, zr_hbm, zh_hbm,
           agg_hbm, xwp_hbm, dinv_hbm,
           src_v, dst_v, dstf_v, hist, iota_v, ddv, dinvv, xwv, xwt,
           bufs, degacc, acc, gsems, ssems):
    c = lax.axis_index("c")
    s = lax.axis_index("s")
    wid = c * NS + s
    pltpu.sync_copy(src_hbm.at[wid], src_v)
    pltpu.sync_copy(dst_hbm.at[wid], dst_v)
    pltpu.sync_copy(dstf_hbm.at[s], dstf_v.at[0])
    pltpu.sync_copy(dstf_hbm.at[NS + s], dstf_v.at[1])
    pltpu.sync_copy(iota_hbm, iota_v)

    @pl.loop(0, NR)
    def _(i):
        hist[i] = jnp.zeros((16,), jnp.float32)

    @pl.when(s == 0)
    def _():
        pltpu.sync_copy(zr_hbm, degacc)

    @pl.when(c == 1)
    def _():
        _tile_rows_copy(zh_hbm, acc, s)

    plsc.subcore_barrier()

    ones16 = jnp.ones((16,), jnp.float32)

    @pl.loop(0, EPW, step=16)
    def _(i):
        for r in range(2):
            idx = dstf_v[r, pl.ds(i, 16)]
            plsc.addupdate_scatter(
                hist, [lax.shift_right_logical(idx, 4), idx & 15], ones16)

    @pl.loop(0, 5)
    def _(r):
        pltpu.sync_copy(hist.at[pl.ds(r * 125, 125)],
                        degacc.at[iota_v.at[r]], add=True)

    plsc.subcore_barrier()

    pltpu.sync_copy(degacc.at[pl.ds(s * RPT, RPT)], ddv.at[pl.ds(0, RPT)])

    @pl.when(s == 0)
    def _():
        pltpu.sync_copy(degacc.at[pl.ds(NR - 1, 1)], ddv.at[pl.ds(RPT, 1)])

    @pl.loop(0, RPT)
    def _(i):
        dinvv[pl.ds(i * 16, 16)] = _rsqrt16(1.0 + ddv[i])

    @pl.when(s == 0)
    def _():
        dinvv[pl.ds(RB, TAILN)] = _rsqrt16(1.0 + ddv[RPT])

    pltpu.sync_copy(xw_hbm.at[pl.ds(s * RB, RB)], xwv)

    @pl.loop(0, RB)
    def _(n):
        spl = plsc.load_gather(dinvv, [jnp.full((16,), 0, jnp.int32) + n])
        xwv[n] = xwv[n] * spl

    pltpu.sync_copy(xwv, xwp_hbm.at[c, pl.ds(s * RB, RB)])
    pltpu.sync_copy(dinvv.at[pl.ds(0, RB)], dinv_hbm.at[pl.ds(s * RB, RB)])

    @pl.when(c == 0)
    def _():
        pltpu.sync_copy(xwv, acc.at[pl.ds(s * RB, RB)])

    @pl.when(s == 0)
    def _():
        pltpu.sync_copy(xw_hbm.at[pl.ds(TAIL0, TAILN)], xwt)

        @pl.loop(0, TAILN)
        def _(n):
            spl = plsc.load_gather(
                dinvv, [jnp.full((16,), RB, jnp.int32) + n])
            xwt[n] = xwt[n] * spl

        pltpu.sync_copy(xwt, xwp_hbm.at[c, pl.ds(TAIL0, TAILN)])
        pltpu.sync_copy(dinvv.at[pl.ds(RB, TAILN)],
                        dinv_hbm.at[pl.ds(TAIL0, TAILN)])

        @pl.when(c == 0)
        def _():
            pltpu.sync_copy(xwt, acc.at[pl.ds(TAIL0, TAILN)])

    plsc.subcore_barrier()

    _edge_pipeline(xwp_hbm.at[c], acc, src_v, dst_v, bufs, gsems, ssems)

    plsc.subcore_barrier()
    _tile_rows_copy(acc, agg_hbm.at[c], s)


@functools.partial(
    pl.kernel,
    out_type=jax.ShapeDtypeStruct((NC, N, D_OUTP), jnp.float32),
    mesh=_MESH,
    scratch_types=[
        pltpu.VMEM((NCH, K), jnp.int32),
        pltpu.VMEM((NCH, K), jnp.int32),
        [pltpu.VMEM((K, D_OUTP), jnp.float32)] * 4,
        pltpu.VMEM_SHARED((N, D_OUTP), jnp.float32),
        [pltpu.SemaphoreType.DMA] * 4,
        [pltpu.SemaphoreType.DMA] * 4,
    ],
    compiler_params=_SC_PARAMS,
)
def _sc_k2(val_hbm, src_hbm, dst_hbm, zero_hbm, out_hbm,
           src_v, dst_v, bufs, acc, gsems, ssems):
    c = lax.axis_index("c")
    s = lax.axis_index("s")
    wid = c * NS + s
    pltpu.sync_copy(src_hbm.at[wid], src_v)
    pltpu.sync_copy(dst_hbm.at[wid], dst_v)

    @pl.when(c == 0)
    def _():
        _tile_rows_copy(val_hbm, acc, s)

    @pl.when(c == 1)
    def _():
        _tile_rows_copy(zero_hbm, acc, s)

    plsc.subcore_barrier()
    _edge_pipeline(val_hbm, acc, src_v, dst_v, bufs, gsems, ssems)
    plsc.subcore_barrier()
    _tile_rows_copy(acc, out_hbm.at[c], s)


_BR = 2000
_G = N // _BR


def _tc_mm_body(x_ref, w1_ref, xw_ref):
    xw_ref[...] = jnp.dot(x_ref[...], w1_ref[...],
                          preferred_element_type=jnp.float32)


def _tc_mm(x, w1):
    return pl.pallas_call(
        _tc_mm_body,
        grid=(_G,),
        in_specs=[
            pl.BlockSpec((_BR, D_IN), lambda i: (i, 0)),
            pl.BlockSpec((D_IN, D_HID), lambda i: (0, 0)),
        ],
        out_specs=pl.BlockSpec((_BR, D_HID), lambda i: (i, 0)),
        out_shape=jax.ShapeDtypeStruct((N, D_HID), jnp.float32),
    )(x, w1)


def _tc_b_body(agg_ref, dinv_ref, b1_ref, w2_ref, hwp_ref):
    dinv = jnp.broadcast_to(dinv_ref[...], (_BR, D_HID))
    pre = (agg_ref[0] + agg_ref[1]) * dinv + b1_ref[...]
    h = jnp.where(pre > 0, pre, jnp.exp(jnp.minimum(pre, 0.0)) - 1.0)
    hw = jnp.dot(h, w2_ref[...], preferred_element_type=jnp.float32)
    dinv_o = jnp.broadcast_to(dinv_ref[...], (_BR, D_OUTP))
    hwp_ref[...] = hw * dinv_o


def _tc_b(agg1, dinv, b1, w2p):
    return pl.pallas_call(
        _tc_b_body,
        grid=(_G,),
        in_specs=[
            pl.BlockSpec((NC, _BR, D_HID), lambda i: (0, i, 0)),
            pl.BlockSpec((_BR, 1), lambda i: (i, 0)),
            pl.BlockSpec((1, D_HID), lambda i: (0, 0)),
            pl.BlockSpec((D_HID, D_OUTP), lambda i: (0, 0)),
        ],
        out_specs=pl.BlockSpec((_BR, D_OUTP), lambda i: (i, 0)),
        out_shape=jax.ShapeDtypeStruct((N, D_OUTP), jnp.float32),
    )(agg1, dinv, b1, w2p)


def _tc_c_body(agg_ref, dinv_ref, b2_ref, o_ref):
    dinv_o = jnp.broadcast_to(dinv_ref[...], (_BR, D_OUTP))
    o = (agg_ref[0] + agg_ref[1]) * dinv_o + b2_ref[...]
    col = lax.broadcasted_iota(jnp.int32, (_BR, D_OUTP), 1)
    valid = col < D_OUT
    om = jnp.where(valid, o, jnp.float32(-1e30))
    m = jnp.max(om, axis=1, keepdims=True)
    ex = jnp.where(valid, jnp.exp(o - m), 0.0)
    lse = jnp.log(jnp.sum(ex, axis=1, keepdims=True))
    o_ref[...] = o - m - lse


def _tc_c(agg2, dinv, b2p):
    return pl.pallas_call(
        _tc_c_body,
        grid=(_G,),
        in_specs=[
            pl.BlockSpec((NC, _BR, D_OUTP), lambda i: (0, i, 0)),
            pl.BlockSpec((_BR, 1), lambda i: (i, 0)),
            pl.BlockSpec((1, D_OUTP), lambda i: (0, 0)),
        ],
        out_specs=pl.BlockSpec((_BR, D_OUTP), lambda i: (i, 0)),
        out_shape=jax.ShapeDtypeStruct((N, D_OUTP), jnp.float32),
    )(agg2, dinv, b2p)


@jax.jit
def kernel(node_feature, edge_index, W1, b1, W2, b2):
    src3 = edge_index[0].reshape(NW, NCH, K)
    dst3 = edge_index[1].reshape(NW, NCH, K)
    dstf = edge_index[1].reshape(NW, EPW)
    iota = jnp.arange(NR, dtype=jnp.int32).reshape(5, 125)
    z_r = jnp.zeros((NR, 16), jnp.float32)
    z_h = jnp.zeros((N, D_HID), jnp.float32)
    z_o = jnp.zeros((N, D_OUTP), jnp.float32)
    w2p = jnp.pad(W2, ((0, 0), (0, D_OUTP - D_OUT)))
    b1r = b1.reshape(1, D_HID)
    b2p = jnp.pad(b2, (0, D_OUTP - D_OUT)).reshape(1, D_OUTP)

    xw = _tc_mm(node_feature, W1)
    agg1, _, dinv = _sc_k1(xw, src3, dst3, dstf, iota, z_r, z_h)
    hwp = _tc_b(agg1, dinv.reshape(N, 1), b1r, w2p)
    agg2 = _sc_k2(hwp, src3, dst3, z_o)
    o = _tc_c(agg2, dinv.reshape(N, 1), b2p)
    return o[:, :D_OUT]


# speedup vs baseline: 1.0675x; 1.0675x over previous
import functools

import jax
import jax.numpy as jnp
from jax import lax
from jax.experimental import pallas as pl
from jax.experimental.pallas import tpu as pltpu
from jax.experimental.pallas import tpu_sc as plsc

N = 10000
E = 320000
D_IN = 128
D_HID = 16
D_OUT = 40
D_OUTP = 48

NC = 2
NS = 16
NW = NC * NS
K = 128
NCH = 80
EPW = NCH * K
EP = NW * EPW
NJ = 16
NRH = 640
NR = N // 16
RPT = NR // NS
RB = 16 * RPT
TAIL0 = NS * RB
TAILN = N - TAIL0

_MESH = plsc.VectorSubcoreMesh(core_axis_name="c", subcore_axis_name="s")
_SC_PARAMS = pltpu.CompilerParams(use_tc_tiling_on_sc=False,
                                  needs_layout_passes=False)


def _tile_rows_copy(src, dst, s):
    pltpu.sync_copy(src.at[pl.ds(s * RB, RB)], dst.at[pl.ds(s * RB, RB)])

    @pl.when(s == 0)
    def _():
        pltpu.sync_copy(src.at[pl.ds(TAIL0, TAILN)], dst.at[pl.ds(TAIL0, TAILN)])


def _rsqrt16(x):
    i = plsc.bitcast(x, jnp.int32)
    y = plsc.bitcast(jnp.int32(0x5F3759DF) - lax.shift_right_logical(i, 1),
                     jnp.float32)
    for _ in range(3):
        y = y * (1.5 - 0.5 * x * y * y)
    return y


def _edge_pipeline(val_ref, acc, src_v, dst_v, bufs, gsems, ssems):
    for u in range(4):
        pltpu.async_copy(val_ref.at[src_v.at[u]], bufs[u], gsems[u])

    @pl.loop(0, NCH, step=4)
    def _(j):
        for u in range(4):
            pltpu.make_async_copy(val_ref.at[src_v.at[j + u]], bufs[u],
                                  gsems[u]).wait()
            pltpu.async_copy(bufs[u], acc.at[dst_v.at[j + u]], ssems[u],
                             add=True)
        for u in range(4):
            @pl.when(j + 4 + u < NCH)
            def _(u=u):
                pltpu.make_async_copy(bufs[u], acc.at[dst_v.at[j + u]],
                                      ssems[u]).wait()
                pltpu.async_copy(val_ref.at[src_v.at[j + 4 + u]], bufs[u],
                                 gsems[u])

    for u in range(4):
        pltpu.make_async_copy(bufs[u], acc.at[dst_v.at[0]], ssems[u]).wait()


@functools.partial(
    pl.kernel,
    out_type=(
        jax.ShapeDtypeStruct((NC, N, D_HID), jnp.float32),
        jax.ShapeDtypeStruct((NC, N, D_HID), jnp.float32),
        jax.ShapeDtypeStruct((N,), jnp.float32),
    ),
    mesh=_MESH,
    scratch_types=[
        pltpu.VMEM((NCH, K), jnp.int32),
        pltpu.VMEM((NCH, K), jnp.int32),
        pltpu.VMEM((NCH, K), jnp.int32),
        pltpu.VMEM((NRH, 16), jnp.float32),
        pltpu.VMEM((5, 128), jnp.int32),
        pltpu.VMEM((RPT + 1, 16), jnp.float32),
        pltpu.VMEM((RB + TAILN,), jnp.float32),
        pltpu.VMEM((RB, D_HID), jnp.float32),
        pltpu.VMEM((TAILN, D_HID), jnp.float32),
        [pltpu.VMEM((K, D_HID), jnp.float32)] * 4,
        pltpu.VMEM_SHARED((NRH, 16), jnp.float32),
        pltpu.VMEM_SHARED((N + NJ, D_HID), jnp.float32),
        [pltpu.SemaphoreType.DMA] * 4,
        [pltpu.SemaphoreType.DMA] * 4,
    ],
    compiler_params=_SC_PARAMS,
)
def _sc_k1(xw_hbm, src_hbm, dst_hbm, iota_hbm, zr_hbm, zh_hbm,
           agg_hbm, xwp_hbm, dinv_hbm,
           src_v, dst_v, dsto_v, hist, iota_v, ddv, dinvv, xwv, xwt,
           bufs, degacc, acc, gsems, ssems):
    c = lax.axis_index("c")
    s = lax.axis_index("s")
    wid = c * NS + s
    owid = (1 - c) * NS + s
    pltpu.sync_copy(src_hbm.at[wid], src_v)
    pltpu.sync_copy(dst_hbm.at[wid], dst_v)
    pltpu.sync_copy(dst_hbm.at[owid], dsto_v)
    pltpu.sync_copy(iota_hbm, iota_v)

    @pl.loop(0, NRH)
    def _(i):
        hist[i] = jnp.zeros((16,), jnp.float32)

    @pl.when(s == 0)
    def _():
        pltpu.sync_copy(zr_hbm, degacc)

    @pl.when(c == 1)
    def _():
        _tile_rows_copy(zh_hbm, acc, s)

    plsc.subcore_barrier()

    ones16 = jnp.ones((16,), jnp.float32)

    @pl.loop(0, NCH)
    def _(row):
        for half in (dst_v, dsto_v):
            for cc in range(K // 16):
                idx = half[row, pl.ds(cc * 16, 16)]
                plsc.addupdate_scatter(
                    hist, [lax.shift_right_logical(idx, 4), idx & 15], ones16)

    @pl.loop(0, 5)
    def _(r):
        pltpu.sync_copy(hist.at[pl.ds(r * 128, 128)],
                        degacc.at[iota_v.at[r]], add=True)

    plsc.subcore_barrier()

    pltpu.sync_copy(degacc.at[pl.ds(s * RPT, RPT)], ddv.at[pl.ds(0, RPT)])

    @pl.when(s == 0)
    def _():
        pltpu.sync_copy(degacc.at[pl.ds(NR - 1, 1)], ddv.at[pl.ds(RPT, 1)])

    @pl.loop(0, RPT)
    def _(i):
        dinvv[pl.ds(i * 16, 16)] = _rsqrt16(1.0 + ddv[i])

    @pl.when(s == 0)
    def _():
        dinvv[pl.ds(RB, TAILN)] = _rsqrt16(1.0 + ddv[RPT])

    pltpu.sync_copy(xw_hbm.at[pl.ds(s * RB, RB)], xwv)

    @pl.loop(0, RB)
    def _(n):
        spl = plsc.load_gather(dinvv, [jnp.full((16,), 0, jnp.int32) + n])
        xwv[n] = xwv[n] * spl

    pltpu.sync_copy(xwv, xwp_hbm.at[c, pl.ds(s * RB, RB)])
    pltpu.sync_copy(dinvv.at[pl.ds(0, RB)], dinv_hbm.at[pl.ds(s * RB, RB)])

    @pl.when(c == 0)
    def _():
        pltpu.sync_copy(xwv, acc.at[pl.ds(s * RB, RB)])

    @pl.when(s == 0)
    def _():
        pltpu.sync_copy(xw_hbm.at[pl.ds(TAIL0, TAILN)], xwt)

        @pl.loop(0, TAILN)
        def _(n):
            spl = plsc.load_gather(
                dinvv, [jnp.full((16,), RB, jnp.int32) + n])
            xwt[n] = xwt[n] * spl

        pltpu.sync_copy(xwt, xwp_hbm.at[c, pl.ds(TAIL0, TAILN)])
        pltpu.sync_copy(dinvv.at[pl.ds(RB, TAILN)],
                        dinv_hbm.at[pl.ds(TAIL0, TAILN)])

        @pl.when(c == 0)
        def _():
            pltpu.sync_copy(xwt, acc.at[pl.ds(TAIL0, TAILN)])

    plsc.subcore_barrier()

    _edge_pipeline(xwp_hbm.at[c], acc, src_v, dst_v, bufs, gsems, ssems)

    plsc.subcore_barrier()
    _tile_rows_copy(acc, agg_hbm.at[c], s)


@functools.partial(
    pl.kernel,
    out_type=jax.ShapeDtypeStruct((NC, N, D_OUTP), jnp.float32),
    mesh=_MESH,
    scratch_types=[
        pltpu.VMEM((NCH, K), jnp.int32),
        pltpu.VMEM((NCH, K), jnp.int32),
        [pltpu.VMEM((K, D_OUTP), jnp.float32)] * 4,
        pltpu.VMEM_SHARED((N + NJ, D_OUTP), jnp.float32),
        [pltpu.SemaphoreType.DMA] * 4,
        [pltpu.SemaphoreType.DMA] * 4,
    ],
    compiler_params=_SC_PARAMS,
)
def _sc_k2(val_hbm, src_hbm, dst_hbm, zero_hbm, out_hbm,
           src_v, dst_v, bufs, acc, gsems, ssems):
    c = lax.axis_index("c")
    s = lax.axis_index("s")
    wid = c * NS + s
    pltpu.sync_copy(src_hbm.at[wid], src_v)
    pltpu.sync_copy(dst_hbm.at[wid], dst_v)

    @pl.when(c == 0)
    def _():
        _tile_rows_copy(val_hbm, acc, s)

    @pl.when(c == 1)
    def _():
        _tile_rows_copy(zero_hbm, acc, s)

    plsc.subcore_barrier()
    _edge_pipeline(val_hbm, acc, src_v, dst_v, bufs, gsems, ssems)
    plsc.subcore_barrier()
    _tile_rows_copy(acc, out_hbm.at[c], s)


_BR = 2000
_G = N // _BR


def _tc_mm_body(x_ref, w1_ref, xw_ref):
    xw_ref[...] = jnp.dot(x_ref[...], w1_ref[...],
                          preferred_element_type=jnp.float32)


def _tc_mm(x, w1):
    return pl.pallas_call(
        _tc_mm_body,
        grid=(_G,),
        in_specs=[
            pl.BlockSpec((_BR, D_IN), lambda i: (i, 0)),
            pl.BlockSpec((D_IN, D_HID), lambda i: (0, 0)),
        ],
        out_specs=pl.BlockSpec((_BR, D_HID), lambda i: (i, 0)),
        out_shape=jax.ShapeDtypeStruct((N, D_HID), jnp.float32),
    )(x, w1)


def _tc_b_body(agg_ref, dinv_ref, b1_ref, w2_ref, hwp_ref):
    dinv = jnp.broadcast_to(dinv_ref[...], (_BR, D_HID))
    pre = (agg_ref[0] + agg_ref[1]) * dinv + b1_ref[...]
    h = jnp.where(pre > 0, pre, jnp.exp(jnp.minimum(pre, 0.0)) - 1.0)
    hw = jnp.dot(h, w2_ref[...], preferred_element_type=jnp.float32)
    dinv_o = jnp.broadcast_to(dinv_ref[...], (_BR, D_OUTP))
    hwp_ref[...] = hw * dinv_o


def _tc_b(agg1, dinv, b1, w2p):
    return pl.pallas_call(
        _tc_b_body,
        grid=(_G,),
        in_specs=[
            pl.BlockSpec((NC, _BR, D_HID), lambda i: (0, i, 0)),
            pl.BlockSpec((_BR, 1), lambda i: (i, 0)),
            pl.BlockSpec((1, D_HID), lambda i: (0, 0)),
            pl.BlockSpec((D_HID, D_OUTP), lambda i: (0, 0)),
        ],
        out_specs=pl.BlockSpec((_BR, D_OUTP), lambda i: (i, 0)),
        out_shape=jax.ShapeDtypeStruct((N, D_OUTP), jnp.float32),
    )(agg1, dinv, b1, w2p)


def _tc_c_body(agg_ref, dinv_ref, b2_ref, o_ref):
    dinv_o = jnp.broadcast_to(dinv_ref[...], (_BR, D_OUTP))
    o = (agg_ref[0] + agg_ref[1]) * dinv_o + b2_ref[...]
    col = lax.broadcasted_iota(jnp.int32, (_BR, D_OUTP), 1)
    valid = col < D_OUT
    om = jnp.where(valid, o, jnp.float32(-1e30))
    m = jnp.max(om, axis=1, keepdims=True)
    ex = jnp.where(valid, jnp.exp(o - m), 0.0)
    lse = jnp.log(jnp.sum(ex, axis=1, keepdims=True))
    o_ref[...] = o - m - lse


def _tc_c(agg2, dinv, b2p):
    return pl.pallas_call(
        _tc_c_body,
        grid=(_G,),
        in_specs=[
            pl.BlockSpec((NC, _BR, D_OUTP), lambda i: (0, i, 0)),
            pl.BlockSpec((_BR, 1), lambda i: (i, 0)),
            pl.BlockSpec((1, D_OUTP), lambda i: (0, 0)),
        ],
        out_specs=pl.BlockSpec((_BR, D_OUTP), lambda i: (i, 0)),
        out_shape=jax.ShapeDtypeStruct((N, D_OUTP), jnp.float32),
    )(agg2, dinv, b2p)


@jax.jit
def kernel(node_feature, edge_index, W1, b1, W2, b2):
    di = jnp.arange(EP - E, dtype=jnp.int32)
    srcp = jnp.concatenate([edge_index[0], di % N]).reshape(NW, NCH, K)
    dstp = jnp.concatenate([edge_index[1], N + (di % NJ)]).reshape(NW, NCH, K)
    iota = jnp.arange(NRH, dtype=jnp.int32).reshape(5, 128)
    z_r = jnp.zeros((NRH, 16), jnp.float32)
    z_h = jnp.zeros((N, D_HID), jnp.float32)
    z_o = jnp.zeros((N, D_OUTP), jnp.float32)
    w2p = jnp.pad(W2, ((0, 0), (0, D_OUTP - D_OUT)))
    b1r = b1.reshape(1, D_HID)
    b2p = jnp.pad(b2, (0, D_OUTP - D_OUT)).reshape(1, D_OUTP)

    xw = _tc_mm(node_feature, W1)
    agg1, _, dinv = _sc_k1(xw, srcp, dstp, iota, z_r, z_h)
    hwp = _tc_b(agg1, dinv.reshape(N, 1), b1r, w2p)
    agg2 = _sc_k2(hwp, srcp, dstp, z_o)
    o = _tc_c(agg2, dinv.reshape(N, 1), b2p)
    return o[:, :D_OUT]


# speedup vs baseline: 1.1361x; 1.0643x over previous
import functools

import jax
import jax.numpy as jnp
from jax import lax
from jax.experimental import pallas as pl
from jax.experimental.pallas import tpu as pltpu
from jax.experimental.pallas import tpu_sc as plsc

N = 10000
E = 320000
D_IN = 128
D_HID = 16
D_OUT = 40
D_OUTP = 48

NC = 2
NS = 16
NW = NC * NS
K = 128
NCH = 80
EPW = NCH * K
EP = NW * EPW
NJ = 16
NRH = 640
NR = N // 16
RPT = NR // NS
RB = 16 * RPT
TAIL0 = NS * RB
TAILN = N - TAIL0

_MESH = plsc.VectorSubcoreMesh(core_axis_name="c", subcore_axis_name="s")
_SC_PARAMS = pltpu.CompilerParams(use_tc_tiling_on_sc=False,
                                  needs_layout_passes=False)


def _tile_rows_copy(src, dst, s):
    pltpu.sync_copy(src.at[pl.ds(s * RB, RB)], dst.at[pl.ds(s * RB, RB)])

    @pl.when(s == 0)
    def _():
        pltpu.sync_copy(src.at[pl.ds(TAIL0, TAILN)], dst.at[pl.ds(TAIL0, TAILN)])


def _rsqrt16(x):
    i = plsc.bitcast(x, jnp.int32)
    y = plsc.bitcast(jnp.int32(0x5F3759DF) - lax.shift_right_logical(i, 1),
                     jnp.float32)
    for _ in range(3):
        y = y * (1.5 - 0.5 * x * y * y)
    return y


def _edge_pipeline(val_ref, acc, src_v, dst_v, bufs, gsems, ssems):
    for u in range(4):
        pltpu.async_copy(val_ref.at[src_v.at[u]], bufs[u], gsems[u])

    @pl.loop(0, NCH, step=4)
    def _(j):
        for u in range(4):
            pltpu.make_async_copy(val_ref.at[src_v.at[j + u]], bufs[u],
                                  gsems[u]).wait()
            pltpu.async_copy(bufs[u], acc.at[dst_v.at[j + u]], ssems[u],
                             add=True)
        for u in range(4):
            @pl.when(j + 4 + u < NCH)
            def _(u=u):
                pltpu.make_async_copy(bufs[u], acc.at[dst_v.at[j + u]],
                                      ssems[u]).wait()
                pltpu.async_copy(val_ref.at[src_v.at[j + 4 + u]], bufs[u],
                                 gsems[u])

    for u in range(4):
        pltpu.make_async_copy(bufs[u], acc.at[dst_v.at[0]], ssems[u]).wait()


NPR = N // 8
PB = RB // 8

@functools.partial(
    pl.kernel,
    out_type=(
        jax.ShapeDtypeStruct((NC, NPR, 128), jnp.float32),
        jax.ShapeDtypeStruct((NC, N, D_HID), jnp.float32),
        jax.ShapeDtypeStruct((NPR, 384), jnp.float32),
        jax.ShapeDtypeStruct((N,), jnp.float32),
    ),
    mesh=_MESH,
    scratch_types=[
        pltpu.VMEM((NCH, K), jnp.int32),
        pltpu.VMEM((NCH, K), jnp.int32),
        pltpu.VMEM((NCH, K), jnp.int32),
        pltpu.VMEM((NRH, 16), jnp.float32),
        pltpu.VMEM((5, 128), jnp.int32),
        pltpu.VMEM((RPT + 1, 16), jnp.float32),
        pltpu.VMEM((RB + TAILN,), jnp.float32),
        pltpu.VMEM((RB, D_HID), jnp.float32),
        pltpu.VMEM((TAILN, D_HID), jnp.float32),
        pltpu.VMEM((PB, 128), jnp.float32),
        pltpu.VMEM((PB, 384), jnp.float32),
        pltpu.VMEM((2, 128), jnp.float32),
        pltpu.VMEM((2, 384), jnp.float32),
        [pltpu.VMEM((K, D_HID), jnp.float32)] * 4,
        pltpu.VMEM_SHARED((NRH, 16), jnp.float32),
        pltpu.VMEM_SHARED((N + NJ, D_HID), jnp.float32),
        [pltpu.SemaphoreType.DMA] * 4,
        [pltpu.SemaphoreType.DMA] * 4,
    ],
    compiler_params=_SC_PARAMS,
)
def _sc_k1(xw_hbm, src_hbm, dst_hbm, iota_hbm,
           agg_hbm, xwp_hbm, dinvr48_hbm, dinvf_hbm,
           src_v, dst_v, dsto_v, hist, iota_v, ddv, dinvv, xwv, xwt,
           dinvr_v, dinvr48_v, dinvrt, dinvr48t,
           bufs, degacc, acc, gsems, ssems):
    c = lax.axis_index("c")
    s = lax.axis_index("s")
    wid = c * NS + s
    owid = (1 - c) * NS + s
    pltpu.sync_copy(src_hbm.at[wid], src_v)
    pltpu.sync_copy(dst_hbm.at[wid], dst_v)
    pltpu.sync_copy(dst_hbm.at[owid], dsto_v)
    pltpu.sync_copy(iota_hbm, iota_v)

    @pl.loop(0, NRH)
    def _(i):
        hist[i] = jnp.zeros((16,), jnp.float32)

    @pl.loop(0, K)
    def _(i):
        bufs[0][i] = jnp.zeros((16,), jnp.float32)

    @pl.when(s == 0)
    def _():
        for q in range(NRH // K):
            pltpu.sync_copy(bufs[0], degacc.at[pl.ds(q * K, K)])

    @pl.when(c == 1)
    def _():
        for q in range(RB // K):
            pltpu.sync_copy(bufs[0], acc.at[pl.ds(s * RB + q * K, K)])
        pltpu.sync_copy(bufs[0].at[pl.ds(0, RB % K)],
                        acc.at[pl.ds(s * RB + (RB // K) * K, RB % K)])

        @pl.when(s == 0)
        def _():
            pltpu.sync_copy(bufs[0].at[pl.ds(0, TAILN)],
                            acc.at[pl.ds(TAIL0, TAILN)])

    plsc.subcore_barrier()

    ones16 = jnp.ones((16,), jnp.float32)

    @pl.loop(0, NCH)
    def _(row):
        for half in (dst_v, dsto_v):
            for cc in range(K // 16):
                idx = half[row, pl.ds(cc * 16, 16)]
                plsc.addupdate_scatter(
                    hist, [lax.shift_right_logical(idx, 4), idx & 15], ones16)

    @pl.loop(0, 5)
    def _(r):
        pltpu.sync_copy(hist.at[pl.ds(r * 128, 128)],
                        degacc.at[iota_v.at[r]], add=True)

    plsc.subcore_barrier()

    pltpu.sync_copy(degacc.at[pl.ds(s * RPT, RPT)], ddv.at[pl.ds(0, RPT)])

    @pl.when(s == 0)
    def _():
        pltpu.sync_copy(degacc.at[pl.ds(NR - 1, 1)], ddv.at[pl.ds(RPT, 1)])

    @pl.loop(0, RPT)
    def _(i):
        dinvv[pl.ds(i * 16, 16)] = _rsqrt16(1.0 + ddv[i])

    @pl.when(s == 0)
    def _():
        dinvv[pl.ds(RB, TAILN)] = _rsqrt16(1.0 + ddv[RPT])

    pltpu.sync_copy(xw_hbm.at[pl.ds(s * RB, RB)], xwv)

    @pl.loop(0, RB)
    def _(n):
        spl = plsc.load_gather(dinvv, [jnp.full((16,), 0, jnp.int32) + n])
        xwv[n] = xwv[n] * spl
        r = lax.shift_right_logical(n, 3)
        l48 = (n & 7) * 48
        for u in range(3):
            dinvr48_v[r, pl.ds(l48 + u * 16, 16)] = spl

    pltpu.sync_copy(xwv, xwp_hbm.at[c, pl.ds(s * RB, RB)])

    @pl.when(c == 0)
    def _():
        pltpu.sync_copy(xwv, acc.at[pl.ds(s * RB, RB)])
        pltpu.sync_copy(dinvr48_v, dinvr48_hbm.at[pl.ds(s * PB, PB)])
        pltpu.sync_copy(dinvv.at[pl.ds(0, RB)], dinvf_hbm.at[pl.ds(s * RB, RB)])

    @pl.when(s == 0)
    def _():
        pltpu.sync_copy(xw_hbm.at[pl.ds(TAIL0, TAILN)], xwt)

        @pl.loop(0, TAILN)
        def _(n):
            spl = plsc.load_gather(
                dinvv, [jnp.full((16,), RB, jnp.int32) + n])
            xwt[n] = xwt[n] * spl
            r = lax.shift_right_logical(n, 3)
            l48 = (n & 7) * 48
            for u in range(3):
                dinvr48t[r, pl.ds(l48 + u * 16, 16)] = spl

        pltpu.sync_copy(xwt, xwp_hbm.at[c, pl.ds(TAIL0, TAILN)])

        @pl.when(c == 0)
        def _():
            pltpu.sync_copy(xwt, acc.at[pl.ds(TAIL0, TAILN)])
            pltpu.sync_copy(dinvr48t, dinvr48_hbm.at[pl.ds(NS * PB, 2)])
            pltpu.sync_copy(dinvv.at[pl.ds(RB, TAILN)],
                            dinvf_hbm.at[pl.ds(TAIL0, TAILN)])

    plsc.subcore_barrier()

    _edge_pipeline(xwp_hbm.at[c], acc, src_v, dst_v, bufs, gsems, ssems)

    plsc.subcore_barrier()
    pltpu.sync_copy(acc.at[pl.ds(s * RB, RB)], xwv)

    @pl.loop(0, RB)
    def _(n):
        spl = plsc.load_gather(dinvv, [jnp.full((16,), 0, jnp.int32) + n])
        dinvr_v[lax.shift_right_logical(n, 3),
                pl.ds((n & 7) * 16, 16)] = xwv[n] * spl

    pltpu.sync_copy(dinvr_v, agg_hbm.at[c, pl.ds(s * PB, PB)])

    @pl.when(s == 0)
    def _():
        pltpu.sync_copy(acc.at[pl.ds(TAIL0, TAILN)], xwt)

        @pl.loop(0, TAILN)
        def _(n):
            spl = plsc.load_gather(
                dinvv, [jnp.full((16,), RB, jnp.int32) + n])
            dinvrt[lax.shift_right_logical(n, 3),
                   pl.ds((n & 7) * 16, 16)] = xwt[n] * spl

        pltpu.sync_copy(dinvrt, agg_hbm.at[c, pl.ds(NS * PB, 2)])


@functools.partial(
    pl.kernel,
    out_type=jax.ShapeDtypeStruct((NC, N, D_OUTP), jnp.float32),
    mesh=_MESH,
    scratch_types=[
        pltpu.VMEM((NCH, K), jnp.int32),
        pltpu.VMEM((NCH, K), jnp.int32),
        [pltpu.VMEM((K, D_OUTP), jnp.float32)] * 4,
        pltpu.VMEM_SHARED((N + NJ, D_OUTP), jnp.float32),
        [pltpu.SemaphoreType.DMA] * 4,
        [pltpu.SemaphoreType.DMA] * 4,
    ],
    compiler_params=_SC_PARAMS,
)
def _sc_k2(val_hbm, src_hbm, dst_hbm, zero_hbm, out_hbm,
           src_v, dst_v, bufs, acc, gsems, ssems):
    c = lax.axis_index("c")
    s = lax.axis_index("s")
    wid = c * NS + s
    pltpu.sync_copy(src_hbm.at[wid], src_v)
    pltpu.sync_copy(dst_hbm.at[wid], dst_v)

    @pl.when(c == 0)
    def _():
        _tile_rows_copy(val_hbm, acc, s)

    @pl.when(c == 1)
    def _():
        _tile_rows_copy(zero_hbm, acc, s)

    plsc.subcore_barrier()
    _edge_pipeline(val_hbm, acc, src_v, dst_v, bufs, gsems, ssems)
    plsc.subcore_barrier()
    _tile_rows_copy(acc, out_hbm.at[c], s)


_BR = 2000
_G = N // _BR


def _tc_mm_body(x_ref, w1_ref, xw_ref):
    xw_ref[...] = jnp.dot(x_ref[...], w1_ref[...],
                          preferred_element_type=jnp.float32)


def _tc_mm(x, w1):
    return pl.pallas_call(
        _tc_mm_body,
        grid=(_G,),
        in_specs=[
            pl.BlockSpec((_BR, D_IN), lambda i: (i, 0)),
            pl.BlockSpec((D_IN, D_HID), lambda i: (0, 0)),
        ],
        out_specs=pl.BlockSpec((_BR, D_HID), lambda i: (i, 0)),
        out_shape=jax.ShapeDtypeStruct((N, D_HID), jnp.float32),
    )(x, w1)


def _tc_b_body(agg_ref, b1_ref, w2bd_ref, dinvr48_ref, hwp_ref):
    pre = agg_ref[0] + agg_ref[1] + b1_ref[...]
    h = jnp.where(pre > 0, pre, jnp.exp(jnp.minimum(pre, 0.0)) - 1.0)
    hw = jnp.dot(h, w2bd_ref[...], preferred_element_type=jnp.float32)
    hwp_ref[...] = hw * dinvr48_ref[...]


def _tc_b(agg1, b1t, w2bd, dinvr48):
    return pl.pallas_call(
        _tc_b_body,
        grid=(1,),
        in_specs=[
            pl.BlockSpec((NC, NPR, 128), lambda i: (0, 0, 0)),
            pl.BlockSpec((1, 128), lambda i: (0, 0)),
            pl.BlockSpec((128, 384), lambda i: (0, 0)),
            pl.BlockSpec((NPR, 384), lambda i: (0, 0)),
        ],
        out_specs=pl.BlockSpec((NPR, 384), lambda i: (0, 0)),
        out_shape=jax.ShapeDtypeStruct((NPR, 384), jnp.float32),
    )(agg1, b1t, w2bd, dinvr48)


def _tc_c_body(agg_ref, dinv_ref, b2_ref, o_ref):
    dinv = jnp.broadcast_to(dinv_ref[...], (N, D_OUTP))
    o = (agg_ref[0] + agg_ref[1]) * dinv + b2_ref[...]
    col = lax.broadcasted_iota(jnp.int32, (N, D_OUTP), 1)
    valid = col < D_OUT
    om = jnp.where(valid, o, jnp.float32(-1e30))
    m = jnp.max(om, axis=1, keepdims=True)
    ex = jnp.where(valid, jnp.exp(o - m), 0.0)
    lse = jnp.log(jnp.sum(ex, axis=1, keepdims=True))
    o_ref[...] = o - m - lse


def _tc_c(agg2, dinv, b2p):
    return pl.pallas_call(
        _tc_c_body,
        grid=(1,),
        in_specs=[
            pl.BlockSpec((NC, N, D_OUTP), lambda i: (0, 0, 0)),
            pl.BlockSpec((N, 1), lambda i: (0, 0)),
            pl.BlockSpec((1, D_OUTP), lambda i: (0, 0)),
        ],
        out_specs=pl.BlockSpec((N, D_OUTP), lambda i: (0, 0)),
        out_shape=jax.ShapeDtypeStruct((N, D_OUTP), jnp.float32),
    )(agg2, dinv, b2p)


@jax.jit
def kernel(node_feature, edge_index, W1, b1, W2, b2):
    di = jnp.arange(EP - E, dtype=jnp.int32)
    srcp = jnp.concatenate([edge_index[0], di % N]).reshape(NW, NCH, K)
    dstp = jnp.concatenate([edge_index[1], N + (di % NJ)]).reshape(NW, NCH, K)
    iota = jnp.arange(NRH, dtype=jnp.int32).reshape(5, 128)
    w2p = jnp.pad(W2, ((0, 0), (0, D_OUTP - D_OUT)))
    w2bd = jnp.zeros((128, 384), jnp.float32)
    for u in range(8):
        w2bd = lax.dynamic_update_slice(w2bd, w2p, (u * D_HID, u * D_OUTP))
    b1t = jnp.tile(b1, 8).reshape(1, 128)
    b2p = jnp.pad(b2, (0, D_OUTP - D_OUT)).reshape(1, D_OUTP)

    xw = _tc_mm(node_feature, W1)
    agg1, _, dinvr48, dinvf = _sc_k1(xw, srcp, dstp, iota)
    hwp = _tc_b(agg1, b1t, w2bd, dinvr48)
    z_o = jnp.zeros((N, D_OUTP), jnp.float32)
    agg2 = _sc_k2(hwp.reshape(N, D_OUTP), srcp, dstp, z_o)
    o = _tc_c(agg2, dinvf.reshape(N, 1), b2p)
    return o[:, :D_OUT]


# speedup vs baseline: 1.1463x; 1.0089x over previous
import functools

import jax
import jax.numpy as jnp
from jax import lax
from jax.experimental import pallas as pl
from jax.experimental.pallas import tpu as pltpu
from jax.experimental.pallas import tpu_sc as plsc

N = 10000
E = 320000
D_IN = 128
D_HID = 16
D_OUT = 40
D_OUTP = 48

NC = 2
NS = 16
NW = NC * NS
K = 128
NCH = 80
EPW = NCH * K
EP = NW * EPW
NJ = 16
NRH = 640
NR = N // 16
RPT = NR // NS
RB = 16 * RPT
TAIL0 = NS * RB
TAILN = N - TAIL0

_MESH = plsc.VectorSubcoreMesh(core_axis_name="c", subcore_axis_name="s")
_SC_PARAMS = pltpu.CompilerParams(use_tc_tiling_on_sc=False,
                                  needs_layout_passes=False)


def _tile_rows_copy(src, dst, s):
    pltpu.sync_copy(src.at[pl.ds(s * RB, RB)], dst.at[pl.ds(s * RB, RB)])

    @pl.when(s == 0)
    def _():
        pltpu.sync_copy(src.at[pl.ds(TAIL0, TAILN)], dst.at[pl.ds(TAIL0, TAILN)])


def _rsqrt16(x):
    i = plsc.bitcast(x, jnp.int32)
    y = plsc.bitcast(jnp.int32(0x5F3759DF) - lax.shift_right_logical(i, 1),
                     jnp.float32)
    for _ in range(3):
        y = y * (1.5 - 0.5 * x * y * y)
    return y


def _edge_pipeline(val_ref, acc, src_v, dst_v, bufs, gsems, ssems):
    for u in range(4):
        pltpu.async_copy(val_ref.at[src_v.at[u]], bufs[u], gsems[u])

    @pl.loop(0, NCH, step=4)
    def _(j):
        for u in range(4):
            pltpu.make_async_copy(val_ref.at[src_v.at[j + u]], bufs[u],
                                  gsems[u]).wait()
            pltpu.async_copy(bufs[u], acc.at[dst_v.at[j + u]], ssems[u],
                             add=True)
        for u in range(4):
            @pl.when(j + 4 + u < NCH)
            def _(u=u):
                pltpu.make_async_copy(bufs[u], acc.at[dst_v.at[j + u]],
                                      ssems[u]).wait()
                pltpu.async_copy(val_ref.at[src_v.at[j + 4 + u]], bufs[u],
                                 gsems[u])

    for u in range(4):
        pltpu.make_async_copy(bufs[u], acc.at[dst_v.at[0]], ssems[u]).wait()


NPR = N // 8
PB = RB // 8

@functools.partial(
    pl.kernel,
    out_type=(
        jax.ShapeDtypeStruct((NC, NPR, 128), jnp.float32),
        jax.ShapeDtypeStruct((NC, N, D_HID), jnp.float32),
        jax.ShapeDtypeStruct((NPR, 384), jnp.float32),
        jax.ShapeDtypeStruct((N,), jnp.float32),
    ),
    mesh=_MESH,
    scratch_types=[
        pltpu.VMEM((NCH, K), jnp.int32),
        pltpu.VMEM((NCH, K), jnp.int32),
        pltpu.VMEM((NCH, K), jnp.int32),
        pltpu.VMEM((NRH, 16), jnp.float32),
        pltpu.VMEM((5, 128), jnp.int32),
        pltpu.VMEM((RPT + 1, 16), jnp.float32),
        pltpu.VMEM((RB + TAILN,), jnp.float32),
        pltpu.VMEM((RB, D_HID), jnp.float32),
        pltpu.VMEM((TAILN, D_HID), jnp.float32),
        pltpu.VMEM((PB, 128), jnp.float32),
        pltpu.VMEM((PB, 384), jnp.float32),
        pltpu.VMEM((2, 128), jnp.float32),
        pltpu.VMEM((2, 384), jnp.float32),
        [pltpu.VMEM((K, D_HID), jnp.float32)] * 4,
        pltpu.VMEM_SHARED((NRH, 16), jnp.float32),
        pltpu.VMEM_SHARED((N + NJ, D_HID), jnp.float32),
        [pltpu.SemaphoreType.DMA] * 4,
        [pltpu.SemaphoreType.DMA] * 4,
    ],
    compiler_params=_SC_PARAMS,
)
def _sc_k1(xw_hbm, src_hbm, dst_hbm, iota_hbm,
           agg_hbm, xwp_hbm, dinvr48_hbm, dinvf_hbm,
           src_v, dst_v, dsto_v, hist, iota_v, ddv, dinvv, xwv, xwt,
           dinvr_v, dinvr48_v, dinvrt, dinvr48t,
           bufs, degacc, acc, gsems, ssems):
    c = lax.axis_index("c")
    s = lax.axis_index("s")
    wid = c * NS + s
    owid = (1 - c) * NS + s
    pltpu.sync_copy(src_hbm.at[wid], src_v)
    pltpu.sync_copy(dst_hbm.at[wid], dst_v)
    pltpu.sync_copy(dst_hbm.at[owid], dsto_v)
    pltpu.sync_copy(iota_hbm, iota_v)

    @pl.loop(0, NRH)
    def _(i):
        hist[i] = jnp.zeros((16,), jnp.float32)

    @pl.loop(0, K)
    def _(i):
        bufs[0][i] = jnp.zeros((16,), jnp.float32)

    @pl.when(s == 0)
    def _():
        for q in range(NRH // K):
            pltpu.sync_copy(bufs[0], degacc.at[pl.ds(q * K, K)])

    @pl.when(c == 1)
    def _():
        for q in range(RB // K):
            pltpu.sync_copy(bufs[0], acc.at[pl.ds(s * RB + q * K, K)])
        pltpu.sync_copy(bufs[0].at[pl.ds(0, RB % K)],
                        acc.at[pl.ds(s * RB + (RB // K) * K, RB % K)])

        @pl.when(s == 0)
        def _():
            pltpu.sync_copy(bufs[0].at[pl.ds(0, TAILN)],
                            acc.at[pl.ds(TAIL0, TAILN)])

    plsc.subcore_barrier()

    ones16 = jnp.ones((16,), jnp.float32)

    @pl.loop(0, NCH)
    def _(row):
        for half in (dst_v, dsto_v):
            for cc in range(K // 16):
                idx = half[row, pl.ds(cc * 16, 16)]
                plsc.addupdate_scatter(
                    hist, [lax.shift_right_logical(idx, 4), idx & 15], ones16)

    @pl.loop(0, 5)
    def _(r):
        pltpu.sync_copy(hist.at[pl.ds(r * 128, 128)],
                        degacc.at[iota_v.at[r]], add=True)

    plsc.subcore_barrier()

    pltpu.sync_copy(degacc.at[pl.ds(s * RPT, RPT)], ddv.at[pl.ds(0, RPT)])

    @pl.when(s == 0)
    def _():
        pltpu.sync_copy(degacc.at[pl.ds(NR - 1, 1)], ddv.at[pl.ds(RPT, 1)])

    @pl.loop(0, RPT)
    def _(i):
        dinvv[pl.ds(i * 16, 16)] = _rsqrt16(1.0 + ddv[i])

    @pl.when(s == 0)
    def _():
        dinvv[pl.ds(RB, TAILN)] = _rsqrt16(1.0 + ddv[RPT])

    pltpu.sync_copy(xw_hbm.at[pl.ds(s * RB, RB)], xwv)

    @pl.loop(0, RB)
    def _(n):
        spl = plsc.load_gather(dinvv, [jnp.full((16,), 0, jnp.int32) + n])
        xwv[n] = xwv[n] * spl
        r = lax.shift_right_logical(n, 3)
        l48 = (n & 7) * 48
        for u in range(3):
            dinvr48_v[r, pl.ds(l48 + u * 16, 16)] = spl

    pltpu.sync_copy(xwv, xwp_hbm.at[c, pl.ds(s * RB, RB)])

    @pl.when(c == 0)
    def _():
        pltpu.sync_copy(xwv, acc.at[pl.ds(s * RB, RB)])
        pltpu.sync_copy(dinvr48_v, dinvr48_hbm.at[pl.ds(s * PB, PB)])
        pltpu.sync_copy(dinvv.at[pl.ds(0, RB)], dinvf_hbm.at[pl.ds(s * RB, RB)])

    @pl.when(s == 0)
    def _():
        pltpu.sync_copy(xw_hbm.at[pl.ds(TAIL0, TAILN)], xwt)

        @pl.loop(0, TAILN)
        def _(n):
            spl = plsc.load_gather(
                dinvv, [jnp.full((16,), RB, jnp.int32) + n])
            xwt[n] = xwt[n] * spl
            r = lax.shift_right_logical(n, 3)
            l48 = (n & 7) * 48
            for u in range(3):
                dinvr48t[r, pl.ds(l48 + u * 16, 16)] = spl

        pltpu.sync_copy(xwt, xwp_hbm.at[c, pl.ds(TAIL0, TAILN)])

        @pl.when(c == 0)
        def _():
            pltpu.sync_copy(xwt, acc.at[pl.ds(TAIL0, TAILN)])
            pltpu.sync_copy(dinvr48t, dinvr48_hbm.at[pl.ds(NS * PB, 2)])
            pltpu.sync_copy(dinvv.at[pl.ds(RB, TAILN)],
                            dinvf_hbm.at[pl.ds(TAIL0, TAILN)])

    plsc.subcore_barrier()

    _edge_pipeline(xwp_hbm.at[c], acc, src_v, dst_v, bufs, gsems, ssems)

    plsc.subcore_barrier()
    pltpu.sync_copy(acc.at[pl.ds(s * RB, RB)], xwv)

    @pl.loop(0, RB)
    def _(n):
        spl = plsc.load_gather(dinvv, [jnp.full((16,), 0, jnp.int32) + n])
        dinvr_v[lax.shift_right_logical(n, 3),
                pl.ds((n & 7) * 16, 16)] = xwv[n] * spl

    pltpu.sync_copy(dinvr_v, agg_hbm.at[c, pl.ds(s * PB, PB)])

    @pl.when(s == 0)
    def _():
        pltpu.sync_copy(acc.at[pl.ds(TAIL0, TAILN)], xwt)

        @pl.loop(0, TAILN)
        def _(n):
            spl = plsc.load_gather(
                dinvv, [jnp.full((16,), RB, jnp.int32) + n])
            dinvrt[lax.shift_right_logical(n, 3),
                   pl.ds((n & 7) * 16, 16)] = xwt[n] * spl

        pltpu.sync_copy(dinvrt, agg_hbm.at[c, pl.ds(NS * PB, 2)])


@functools.partial(
    pl.kernel,
    out_type=jax.ShapeDtypeStruct((NC, N, D_OUTP), jnp.float32),
    mesh=_MESH,
    scratch_types=[
        pltpu.VMEM((NCH, K), jnp.int32),
        pltpu.VMEM((NCH, K), jnp.int32),
        [pltpu.VMEM((K, D_OUTP), jnp.float32)] * 4,
        pltpu.VMEM_SHARED((N + NJ, D_OUTP), jnp.float32),
        [pltpu.SemaphoreType.DMA] * 4,
        [pltpu.SemaphoreType.DMA] * 4,
    ],
    compiler_params=_SC_PARAMS,
)
def _sc_k2(val_hbm, src_hbm, dst_hbm, zero_hbm, out_hbm,
           src_v, dst_v, bufs, acc, gsems, ssems):
    c = lax.axis_index("c")
    s = lax.axis_index("s")
    wid = c * NS + s
    pltpu.sync_copy(src_hbm.at[wid], src_v)
    pltpu.sync_copy(dst_hbm.at[wid], dst_v)

    @pl.when(c == 0)
    def _():
        _tile_rows_copy(val_hbm, acc, s)

    @pl.when(c == 1)
    def _():
        _tile_rows_copy(zero_hbm, acc, s)

    plsc.subcore_barrier()
    _edge_pipeline(val_hbm, acc, src_v, dst_v, bufs, gsems, ssems)
    plsc.subcore_barrier()
    _tile_rows_copy(acc, out_hbm.at[c], s)


_BR = 2000
_G = N // _BR


def _tc_mm_body(x_ref, w1_ref, xw_ref):
    xw_ref[...] = jnp.dot(x_ref[...], w1_ref[...],
                          preferred_element_type=jnp.float32)


def _tc_mm(x, w1):
    return pl.pallas_call(
        _tc_mm_body,
        grid=(1,),
        in_specs=[
            pl.BlockSpec((N, D_IN), lambda i: (0, 0)),
            pl.BlockSpec((D_IN, D_HID), lambda i: (0, 0)),
        ],
        out_specs=pl.BlockSpec((N, D_HID), lambda i: (0, 0)),
        out_shape=jax.ShapeDtypeStruct((N, D_HID), jnp.float32),
    )(x, w1)


def _tc_b_body(agg_ref, b1_ref, w2bd_ref, dinvr48_ref, hwp_ref):
    pre = agg_ref[0] + agg_ref[1] + b1_ref[...]
    h = jnp.where(pre > 0, pre, jnp.exp(jnp.minimum(pre, 0.0)) - 1.0)
    hw = jnp.dot(h, w2bd_ref[...], preferred_element_type=jnp.float32)
    hwp_ref[...] = hw * dinvr48_ref[...]


def _tc_b(agg1, b1t, w2bd, dinvr48):
    return pl.pallas_call(
        _tc_b_body,
        grid=(1,),
        in_specs=[
            pl.BlockSpec((NC, NPR, 128), lambda i: (0, 0, 0)),
            pl.BlockSpec((1, 128), lambda i: (0, 0)),
            pl.BlockSpec((128, 384), lambda i: (0, 0)),
            pl.BlockSpec((NPR, 384), lambda i: (0, 0)),
        ],
        out_specs=pl.BlockSpec((NPR, 384), lambda i: (0, 0)),
        out_shape=jax.ShapeDtypeStruct((NPR, 384), jnp.float32),
    )(agg1, b1t, w2bd, dinvr48)


def _tc_c_body(agg_ref, dinv_ref, b2_ref, o_ref):
    dinv = jnp.broadcast_to(dinv_ref[...], (N, D_OUTP))
    o = (agg_ref[0] + agg_ref[1]) * dinv + b2_ref[...]
    col = lax.broadcasted_iota(jnp.int32, (N, D_OUTP), 1)
    valid = col < D_OUT
    om = jnp.where(valid, o, jnp.float32(-1e30))
    m = jnp.max(om, axis=1, keepdims=True)
    ex = jnp.where(valid, jnp.exp(o - m), 0.0)
    lse = jnp.log(jnp.sum(ex, axis=1, keepdims=True))
    o_ref[...] = o - m - lse


def _tc_c(agg2, dinv, b2p):
    return pl.pallas_call(
        _tc_c_body,
        grid=(1,),
        in_specs=[
            pl.BlockSpec((NC, N, D_OUTP), lambda i: (0, 0, 0)),
            pl.BlockSpec((N, 1), lambda i: (0, 0)),
            pl.BlockSpec((1, D_OUTP), lambda i: (0, 0)),
        ],
        out_specs=pl.BlockSpec((N, D_OUTP), lambda i: (0, 0)),
        out_shape=jax.ShapeDtypeStruct((N, D_OUTP), jnp.float32),
    )(agg2, dinv, b2p)


@jax.jit
def kernel(node_feature, edge_index, W1, b1, W2, b2):
    di = jnp.arange(EP - E, dtype=jnp.int32)
    srcp = jnp.concatenate([edge_index[0], di % N]).reshape(NW, NCH, K)
    dstp = jnp.concatenate([edge_index[1], N + (di % NJ)]).reshape(NW, NCH, K)
    iota = jnp.arange(NRH, dtype=jnp.int32).reshape(5, 128)
    w2p = jnp.pad(W2, ((0, 0), (0, D_OUTP - D_OUT)))
    w2bd = jnp.zeros((128, 384), jnp.float32)
    for u in range(8):
        w2bd = lax.dynamic_update_slice(w2bd, w2p, (u * D_HID, u * D_OUTP))
    b1t = jnp.tile(b1, 8).reshape(1, 128)
    b2p = jnp.pad(b2, (0, D_OUTP - D_OUT)).reshape(1, D_OUTP)

    xw = _tc_mm(node_feature, W1)
    agg1, _, dinvr48, dinvf = _sc_k1(xw, srcp, dstp, iota)
    hwp = _tc_b(agg1, b1t, w2bd, dinvr48)
    z_o = jnp.zeros((N, D_OUTP), jnp.float32)
    agg2 = _sc_k2(hwp.reshape(N, D_OUTP), srcp, dstp, z_o)
    o = _tc_c(agg2, dinvf.reshape(N, 1), b2p)
    return o[:, :D_OUT]
